# baseline (speedup 1.0000x reference)
import functools

import jax
import jax.numpy as jnp
from jax import lax
from jax.experimental import pallas as pl
from jax.experimental.pallas import tpu as pltpu
from jax.experimental.pallas import tpu_sc as plsc

NUM_GENRE = 25
EMBED_DIM = 32
BATCH = 16384

NC = 2
NS = 16
L = 16
NW = NC * NS
BPW = BATCH // NW
CHUNKS = BPW // L


def _sc_body(fea, item_tab, year_tab, genre_wb, out,
             fea_v, idx_v, yidx_v, item_rows, year_rows, genre_rows, w_v,
             sem_i, sem_y):
    wid = lax.axis_index("s") * NC + lax.axis_index("c")
    base = wid * BPW

    pltpu.sync_copy(fea.at[pl.ds(base, BPW), :], fea_v)
    pltpu.sync_copy(genre_wb, w_v)

    def extract(c, carry):
        rid = c * L + lax.iota(jnp.int32, L)
        item_idx = plsc.load_gather(fea_v, [rid, jnp.zeros((L,), jnp.int32)])
        year_idx = plsc.load_gather(fea_v, [rid, jnp.ones((L,), jnp.int32)])
        idx_v[pl.ds(c * L, L)] = item_idx
        yidx_v[pl.ds(c * L, L)] = year_idx
        return carry
    lax.fori_loop(0, CHUNKS, extract, 0)

    cp_i = pltpu.async_copy(item_tab.at[idx_v], item_rows, sem_i)
    cp_y = pltpu.async_copy(year_tab.at[yidx_v], year_rows, sem_y)

    def genre_chunk(c, carry):
        rid = c * L + lax.iota(jnp.int32, L)
        cnt = jnp.zeros((L,), jnp.float32)
        accs = [jnp.zeros((L,), jnp.float32) for _ in range(EMBED_DIM)]
        for g in range(NUM_GENRE):
            bits = plsc.load_gather(
                fea_v, [rid, jnp.full((L,), 2 + g, jnp.int32)]
            ).astype(jnp.float32)
            cnt = cnt + bits
            for j in range(EMBED_DIM):
                accs[j] = accs[j] + bits * w_v[g * EMBED_DIM + j, :]
        inv = 1.0 / jnp.maximum(cnt, 1.0)
        for j in range(EMBED_DIM):
            plsc.store_scatter(
                genre_rows, [rid, jnp.full((L,), j, jnp.int32)], accs[j] * inv)
        return carry
    lax.fori_loop(0, CHUNKS, genre_chunk, 0)

    cp_i.wait()
    cp_y.wait()

    pltpu.sync_copy(item_rows, out.at[pl.ds(base, BPW), pl.ds(0, EMBED_DIM)])
    pltpu.sync_copy(year_rows,
                    out.at[pl.ds(base, BPW), pl.ds(EMBED_DIM, EMBED_DIM)])
    pltpu.sync_copy(genre_rows,
                    out.at[pl.ds(base, BPW), pl.ds(2 * EMBED_DIM, EMBED_DIM)])


@jax.jit
def _run(fea, item_embedding, year_embedding, genre_embedding):
    mesh = plsc.VectorSubcoreMesh(core_axis_name="c", subcore_axis_name="s")
    f = pl.kernel(
        _sc_body,
        out_type=jax.ShapeDtypeStruct((BATCH, 3 * EMBED_DIM), jnp.float32),
        mesh=mesh,
        scratch_types=[
            pltpu.VMEM((BPW, 2 + NUM_GENRE), jnp.int32),
            pltpu.VMEM((BPW,), jnp.int32),
            pltpu.VMEM((BPW,), jnp.int32),
            pltpu.VMEM((BPW, EMBED_DIM), jnp.float32),
            pltpu.VMEM((BPW, EMBED_DIM), jnp.float32),
            pltpu.VMEM((BPW, EMBED_DIM), jnp.float32),
            pltpu.VMEM((NUM_GENRE * EMBED_DIM, L), jnp.float32),
            pltpu.SemaphoreType.DMA,
            pltpu.SemaphoreType.DMA,
        ],
        compiler_params=pltpu.CompilerParams(
            use_tc_tiling_on_sc=False, needs_layout_passes=False),
    )
    return f(fea, item_embedding, year_embedding, genre_embedding)


def kernel(item_fea, item_embedding, year_embedding, genre_embedding):
    wb = jnp.broadcast_to(
        genre_embedding.T.reshape(NUM_GENRE * EMBED_DIM, 1), (NUM_GENRE * EMBED_DIM, L)
    )
    return _run(item_fea.astype(jnp.int32), item_embedding, year_embedding, wb)
